# baseline (speedup 1.0000x reference)
import functools

import jax
import jax.numpy as jnp
from jax import lax
from jax.experimental import pallas as pl
from jax.experimental.pallas import tpu as pltpu
from jax.experimental.pallas import tpu_sc as plsc

_VOCAB = 100000
_MAXLEN = 200
_EMBED = 128
_BATCH = 1024

_NC, _NS = 2, 16
_NW = _NC * _NS
_ROWS = _BATCH * _MAXLEN
_CHUNK = 128
_NCHUNK = _ROWS // _CHUNK
_CPW = _NCHUNK // _NW
_LANES = 16
_DSL = _EMBED // _LANES
_NBUF = 5
_DIST = _NBUF - 2

_mesh = plsc.VectorSubcoreMesh(
    core_axis_name="c", subcore_axis_name="s",
    num_cores=_NC, num_subcores=_NS,
)


@functools.partial(
    pl.kernel,
    out_type=jax.ShapeDtypeStruct((_NCHUNK, _CHUNK, _EMBED), jnp.float32),
    mesh=_mesh,
    scratch_types=(
        [pltpu.VMEM((_MAXLEN, _EMBED), jnp.float32),
         pltpu.VMEM((_CPW, _CHUNK), jnp.int32)]
        + [pltpu.VMEM((_CHUNK, _EMBED), jnp.float32) for _ in range(_NBUF)]
        + [pltpu.SemaphoreType.DMA for _ in range(3 * _NBUF)]
    ),
)
def _embed_kernel(x_hbm, tok_hbm, pos_hbm, out_hbm, pos_v, idxs_v, *rest):
    bufs = rest[:_NBUF]
    gsems = rest[_NBUF:2 * _NBUF]
    hsems = rest[2 * _NBUF:3 * _NBUF]
    wsems = rest[3 * _NBUF:]

    wid = lax.axis_index("s") * _NC + lax.axis_index("c")
    pltpu.sync_copy(pos_hbm, pos_v)
    pltpu.sync_copy(x_hbm.at[wid], idxs_v)

    out_base = wid * _CPW

    _H = _CHUNK // 2

    def start_gather(j, b):
        pltpu.async_copy(tok_hbm.at[idxs_v.at[j, pl.ds(0, _H)]],
                         bufs[b].at[pl.ds(0, _H)], gsems[b])
        pltpu.async_copy(tok_hbm.at[idxs_v.at[j, pl.ds(_H, _H)]],
                         bufs[b].at[pl.ds(_H, _H)], hsems[b])

    def wait_gather_half(j, b, h):
        sem = gsems[b] if h == 0 else hsems[b]
        pltpu.make_async_copy(tok_hbm.at[idxs_v.at[j, pl.ds(h * _H, _H)]],
                              bufs[b].at[pl.ds(h * _H, _H)], sem).wait()

    def start_wb(j, b):
        pltpu.async_copy(bufs[b], out_hbm.at[out_base + j], wsems[b])

    def wait_wb(j, b):
        pltpu.make_async_copy(bufs[b], out_hbm.at[out_base + j], wsems[b]).wait()

    def add_pos_half(j, b, h):
        base_mod = lax.rem((out_base + j) * _CHUNK, _MAXLEN)

        @plsc.parallel_loop(h * _H, (h + 1) * _H)
        def _add(r):
            pr0 = base_mod + r
            pr = pr0 - jnp.where(pr0 >= _MAXLEN, _MAXLEN, 0)
            for d in range(_DSL):
                sl = pl.ds(d * _LANES, _LANES)
                plsc.addupdate(bufs[b].at[r, sl], pos_v[pr, sl])

    def process(j, b):
        wait_gather_half(j, b, 0)
        add_pos_half(j, b, 0)
        wait_gather_half(j, b, 1)
        add_pos_half(j, b, 1)

    for j in range(_DIST):
        start_gather(j, j)

    _NFULL = ((_CPW - _DIST) // _NBUF) * _NBUF

    def body(k, carry):
        for b in range(_NBUF):
            j = _NBUF * k + b

            nb = (b + _DIST) % _NBUF

            @pl.when(j >= 2)
            def _drain():
                wait_wb(j - 2, nb)

            start_gather(j + _DIST, nb)

            process(j, b)
            start_wb(j, b)
        return carry

    lax.fori_loop(0, _NFULL // _NBUF, body, 0)

    for j in range(_NFULL, _CPW):
        b = j % _NBUF
        if j + _DIST < _CPW:
            nb = (j + _DIST) % _NBUF
            wait_wb(j - 2, nb)
            start_gather(j + _DIST, nb)
        process(j, b)
        start_wb(j, b)

    for j in range(_CPW - _NBUF, _CPW):
        wait_wb(j, j % _NBUF)


def kernel(x, token_table, pos_table):
    x3 = x.astype(jnp.int32).reshape(_NW, _CPW, _CHUNK)
    out = _embed_kernel(x3, token_table, pos_table)
    return out.reshape(_BATCH, _MAXLEN, _EMBED)


# speedup vs baseline: 1.0192x; 1.0192x over previous
import functools

import jax
import jax.numpy as jnp
from jax import lax
from jax.experimental import pallas as pl
from jax.experimental.pallas import tpu as pltpu
from jax.experimental.pallas import tpu_sc as plsc

_VOCAB = 100000
_MAXLEN = 200
_EMBED = 128
_BATCH = 1024

_NC, _NS = 2, 16
_NW = _NC * _NS
_ROWS = _BATCH * _MAXLEN
_CHUNK = 128
_NCHUNK = _ROWS // _CHUNK
_CPW = _NCHUNK // _NW
_LANES = 16
_DSL = _EMBED // _LANES
_NBUF = 5
_DIST = _NBUF - 2

_mesh = plsc.VectorSubcoreMesh(
    core_axis_name="c", subcore_axis_name="s",
    num_cores=_NC, num_subcores=_NS,
)


@functools.partial(
    pl.kernel,
    out_type=jax.ShapeDtypeStruct((_NCHUNK, _CHUNK, _EMBED), jnp.float32),
    mesh=_mesh,
    scratch_types=(
        [pltpu.VMEM((_MAXLEN, _EMBED), jnp.float32),
         pltpu.VMEM((_CPW, _CHUNK), jnp.int32)]
        + [pltpu.VMEM((_CHUNK, _EMBED), jnp.float32) for _ in range(_NBUF)]
        + [pltpu.SemaphoreType.DMA for _ in range(2 * _NBUF)]
    ),
)
def _embed_kernel(x_hbm, tok_hbm, pos_hbm, out_hbm, pos_v, idxs_v, *rest):
    bufs = rest[:_NBUF]
    gsems = rest[_NBUF:2 * _NBUF]
    wsems = rest[2 * _NBUF:]

    wid = lax.axis_index("s") * _NC + lax.axis_index("c")
    pltpu.sync_copy(pos_hbm, pos_v)
    pltpu.sync_copy(x_hbm.at[wid], idxs_v)

    out_base = wid * _CPW

    def start_gather(j, b):
        pltpu.async_copy(tok_hbm.at[idxs_v.at[j]], bufs[b], gsems[b])

    def wait_gather(j, b):
        pltpu.make_async_copy(tok_hbm.at[idxs_v.at[j]], bufs[b], gsems[b]).wait()

    def start_wb(j, b):
        pltpu.async_copy(bufs[b], out_hbm.at[out_base + j], wsems[b])

    def wait_wb(j, b):
        pltpu.make_async_copy(bufs[b], out_hbm.at[out_base + j], wsems[b]).wait()

    def add_pos(j, b):
        base_mod = lax.rem((out_base + j) * _CHUNK, _MAXLEN)

        @plsc.parallel_loop(0, _CHUNK)
        def _add(r):
            pr0 = base_mod + r
            pr = pr0 - jnp.where(pr0 >= _MAXLEN, _MAXLEN, 0)
            for d in range(_DSL):
                sl = pl.ds(d * _LANES, _LANES)
                plsc.addupdate(bufs[b].at[r, sl], pos_v[pr, sl])

    for j in range(_DIST):
        start_gather(j, j)

    _NFULL = ((_CPW - _DIST) // _NBUF) * _NBUF

    def body(k, carry):
        for b in range(_NBUF):
            j = _NBUF * k + b

            nb = (b + _DIST) % _NBUF

            @pl.when(j >= 2)
            def _drain():
                wait_wb(j - 2, nb)

            start_gather(j + _DIST, nb)

            wait_gather(j, b)
            add_pos(j, b)
            start_wb(j, b)
        return carry

    lax.fori_loop(0, _NFULL // _NBUF, body, 0)

    for j in range(_NFULL, _CPW):
        b = j % _NBUF
        if j + _DIST < _CPW:
            nb = (j + _DIST) % _NBUF
            wait_wb(j - 2, nb)
            start_gather(j + _DIST, nb)
        wait_gather(j, b)
        add_pos(j, b)
        start_wb(j, b)

    for j in range(_CPW - _NBUF, _CPW):
        wait_wb(j, j % _NBUF)


def kernel(x, token_table, pos_table):
    x3 = x.astype(jnp.int32).reshape(_NW, _CPW, _CHUNK)
    out = _embed_kernel(x3, token_table, pos_table)
    return out.reshape(_BATCH, _MAXLEN, _EMBED)
